# baseline (speedup 1.0000x reference)
import functools

import jax
import jax.numpy as jnp
from jax.experimental import pallas as pl
from jax.experimental.pallas import tpu as pltpu

BM = 512
BN = 2048


def _mm_kernel(scale_ref, x_ref, w_ref, o_ref):
    acc = jax.lax.dot_general(
        x_ref[...].astype(jnp.bfloat16),
        w_ref[...].astype(jnp.bfloat16),
        (((1,), (1,)), ((), ())),
        preferred_element_type=jnp.float32,
    )
    o_ref[...] = acc * scale_ref[0]


@jax.jit
def kernel(x, W, scale):
    M, K = x.shape
    N = W.shape[0]
    grid = (N // BN, M // BM)
    return pl.pallas_call(
        _mm_kernel,
        grid_spec=pltpu.PrefetchScalarGridSpec(
            num_scalar_prefetch=1,
            grid=grid,
            in_specs=[
                pl.BlockSpec((BM, K), lambda n, m, *_: (m, 0)),
                pl.BlockSpec((BN, K), lambda n, m, *_: (n, 0)),
            ],
            out_specs=pl.BlockSpec((BM, BN), lambda n, m, *_: (m, n)),
        ),
        out_shape=jax.ShapeDtypeStruct((M, N), jnp.float32),
    )(scale, x, W)


# speedup vs baseline: 1.2174x; 1.2174x over previous
import jax
import jax.numpy as jnp
from jax.experimental import pallas as pl
from jax.experimental.pallas import tpu as pltpu

BN = 512


def _mm_kernel(scale_ref, x_ref, w_ref, o_ref):
    acc = jax.lax.dot_general(
        x_ref[...],
        w_ref[...],
        (((1,), (1,)), ((), ())),
        preferred_element_type=jnp.float32,
    )
    o_ref[...] = acc * scale_ref[0]


@jax.jit
def kernel(x, W, scale):
    M, K = x.shape
    N = W.shape[0]
    return pl.pallas_call(
        _mm_kernel,
        grid_spec=pltpu.PrefetchScalarGridSpec(
            num_scalar_prefetch=1,
            grid=(N // BN,),
            in_specs=[
                pl.BlockSpec((M, K), lambda n, *_: (0, 0)),
                pl.BlockSpec((BN, K), lambda n, *_: (n, 0)),
            ],
            out_specs=pl.BlockSpec((M, BN), lambda n, *_: (0, n)),
        ),
        out_shape=jax.ShapeDtypeStruct((M, N), jnp.float32),
        compiler_params=pltpu.CompilerParams(
            dimension_semantics=("arbitrary",),
        ),
    )(scale, x, W)
